# baseline (speedup 1.0000x reference)
import functools

import jax
import jax.numpy as jnp
from jax import lax
from jax.experimental import pallas as pl
from jax.experimental.pallas import tpu as pltpu
from jax.experimental.pallas import tpu_sc as plsc

N = 10000
E = 160000
ELG = 640000
EMB = 128
DEA = 16
DLGA = 16
C = 16

_PREC = lax.Precision.DEFAULT


def _mm(a, b):
    return lax.dot_general(a, b, (((1,), (0,)), ((), ())),
                           preferred_element_type=jnp.float32,
                           precision=_PREC)


def _k1_body(x_ref, wf_ref, bf_ref, wma_ref, wmb_ref,
             vf_ref, a_ref, b_ref):
    vf = jnp.maximum(_mm(x_ref[...], wf_ref[...]) + bf_ref[...], 0.0)
    vf_ref[...] = vf
    a_ref[...] = _mm(vf, wma_ref[...])
    b_ref[...] = _mm(vf, wmb_ref[...])


def _k1(x, Wf, bf, Wm_a, Wm_b, blk=1000):
    grid = (N // blk,)
    w_spec = pl.BlockSpec((EMB, EMB), lambda i: (0, 0))
    return pl.pallas_call(
        _k1_body,
        grid=grid,
        in_specs=[
            pl.BlockSpec((blk, EMB), lambda i: (i, 0)),
            w_spec,
            pl.BlockSpec((1, EMB), lambda i: (0, 0)),
            w_spec,
            w_spec,
        ],
        out_specs=[
            pl.BlockSpec((blk, EMB), lambda i: (i, 0)),
            pl.BlockSpec((blk, EMB), lambda i: (i, 0)),
            pl.BlockSpec((blk, EMB), lambda i: (i, 0)),
        ],
        out_shape=[
            jax.ShapeDtypeStruct((N, EMB), jnp.float32),
            jax.ShapeDtypeStruct((N, EMB), jnp.float32),
            jax.ShapeDtypeStruct((N, EMB), jnp.float32),
        ],
    )(x, Wf, bf.reshape(1, EMB), Wm_a, Wm_b)


def _k3_body(ab_ref, ea_ref, wme_ref, bm_ref, w1_ref, b1_ref, wmm_ref,
             prev_ref, m_ref):
    me = (ab_ref[...] + _mm(jnp.maximum(ea_ref[...], 0.0), wme_ref[...])
          + bm_ref[...])
    prev_ref[...] = jnp.maximum(_mm(me, w1_ref[...]) + b1_ref[...], 0.0)
    m_ref[...] = _mm(me, wmm_ref[...])


def _k3(ab, edge_attr, Wm_e, bm, W1, b1, Wmsg_m, blk=2000):
    grid = (E // blk,)
    w_spec = pl.BlockSpec((EMB, EMB), lambda i: (0, 0))
    b_spec = pl.BlockSpec((1, EMB), lambda i: (0, 0))
    return pl.pallas_call(
        _k3_body,
        grid=grid,
        in_specs=[
            pl.BlockSpec((blk, EMB), lambda i: (i, 0)),
            pl.BlockSpec((blk, DEA), lambda i: (i, 0)),
            pl.BlockSpec((DEA, EMB), lambda i: (0, 0)),
            b_spec,
            w_spec,
            b_spec,
            w_spec,
        ],
        out_specs=[
            pl.BlockSpec((blk, EMB), lambda i: (i, 0)),
            pl.BlockSpec((blk, EMB), lambda i: (i, 0)),
        ],
        out_shape=[
            jax.ShapeDtypeStruct((E, EMB), jnp.float32),
            jax.ShapeDtypeStruct((E, EMB), jnp.float32),
        ],
    )(ab, edge_attr, Wm_e, bm.reshape(1, EMB), W1, b1.reshape(1, EMB), Wmsg_m)


def _k4_body(lga_ref, wme_ref, g_ref):
    g_ref[...] = _mm(lga_ref[...], wme_ref[...])


def _k4(lg_edge_attr, Wmsg_e, blk=4000):
    grid = (ELG // blk,)
    return pl.pallas_call(
        _k4_body,
        grid=grid,
        in_specs=[
            pl.BlockSpec((blk, DLGA), lambda i: (i, 0)),
            pl.BlockSpec((DLGA, EMB), lambda i: (0, 0)),
        ],
        out_specs=pl.BlockSpec((blk, EMB), lambda i: (i, 0)),
        out_shape=jax.ShapeDtypeStruct((ELG, EMB), jnp.float32),
    )(lg_edge_attr, Wmsg_e)


def _k7_body(prev_ref, nm_ref, ea_ref,
             wr0a_ref, br0a_ref, wr0b_ref, br0b_ref,
             w2_ref, b2_ref,
             wr1a_ref, br1a_ref, wr1b_ref, br1b_ref,
             wr2a_ref, br2a_ref, wr2b_ref, br2b_ref,
             we_ref, be_ref,
             p_ref):
    def res(h, wa, ba, wb, bb):
        h1 = jnp.maximum(_mm(h, wa[...]) + ba[...], 0.0)
        h2 = jnp.maximum(_mm(h1, wb[...]) + bb[...], 0.0)
        return h + h2

    h0 = prev_ref[...] + nm_ref[...]
    xh = res(h0, wr0a_ref, br0a_ref, wr0b_ref, br0b_ref)
    x1 = jnp.maximum(_mm(xh, w2_ref[...]) + b2_ref[...], 0.0)
    x3 = res(x1, wr1a_ref, br1a_ref, wr1b_ref, br1b_ref)
    x4 = res(x3, wr2a_ref, br2a_ref, wr2b_ref, br2b_ref)
    e_emb = _mm(ea_ref[...], we_ref[...]) + be_ref[...]
    p_ref[...] = e_emb * x4


def _k7(prev, new_msg, edge_attr,
        Wr0a, br0a, Wr0b, br0b, W2, b2,
        Wr1a, br1a, Wr1b, br1b, Wr2a, br2a, Wr2b, br2b,
        We, be, blk=2000):
    grid = (E // blk,)
    w_spec = pl.BlockSpec((EMB, EMB), lambda i: (0, 0))
    b_spec = pl.BlockSpec((1, EMB), lambda i: (0, 0))
    return pl.pallas_call(
        _k7_body,
        grid=grid,
        in_specs=[
            pl.BlockSpec((blk, EMB), lambda i: (i, 0)),
            pl.BlockSpec((blk, EMB), lambda i: (i, 0)),
            pl.BlockSpec((blk, DEA), lambda i: (i, 0)),
            w_spec, b_spec, w_spec, b_spec,
            w_spec, b_spec,
            w_spec, b_spec, w_spec, b_spec,
            w_spec, b_spec, w_spec, b_spec,
            pl.BlockSpec((DEA, EMB), lambda i: (0, 0)), b_spec,
        ],
        out_specs=pl.BlockSpec((blk, EMB), lambda i: (i, 0)),
        out_shape=jax.ShapeDtypeStruct((E, EMB), jnp.float32),
    )(prev, new_msg, edge_attr,
      Wr0a, br0a.reshape(1, EMB), Wr0b, br0b.reshape(1, EMB),
      W2, b2.reshape(1, EMB),
      Wr1a, br1a.reshape(1, EMB), Wr1b, br1b.reshape(1, EMB),
      Wr2a, br2a.reshape(1, EMB), Wr2b, br2b.reshape(1, EMB),
      We, be.reshape(1, EMB))


def _k9_body(ne0_ref, ne1_ref, vf_ref, wl0_ref, bl0_ref, w4_ref, w_ref,
             wout_ref, bout_ref, out_ref):
    ne = ne0_ref[...] + ne1_ref[...]
    o1 = jnp.maximum(_mm(ne, wl0_ref[...]) + bl0_ref[...], 0.0)
    o2 = jnp.maximum(_mm(o1, wl0_ref[...]) + bl0_ref[...], 0.0)
    o3 = jnp.maximum(_mm(o2, wl0_ref[...]) + bl0_ref[...], 0.0)
    x5 = _mm(o3, w4_ref[...])
    w0 = w_ref[0, 0]
    w1 = w_ref[0, 1]
    agg = w0 * vf_ref[...] + w1 * x5
    logits = _mm(agg, wout_ref[...]) + bout_ref[...]
    m = jnp.max(logits, axis=-1, keepdims=True)
    s = logits - m
    lse = jnp.log(jnp.sum(jnp.exp(s), axis=-1, keepdims=True))
    out_ref[...] = s - lse


def _k9(ne0, ne1, vf, Wl0, bl0, W4, w01, Wout, bout, blk=1000):
    grid = (N // blk,)
    w_spec = pl.BlockSpec((EMB, EMB), lambda i: (0, 0))
    b_spec = pl.BlockSpec((1, EMB), lambda i: (0, 0))
    return pl.pallas_call(
        _k9_body,
        grid=grid,
        in_specs=[
            pl.BlockSpec((blk, EMB), lambda i: (i, 0)),
            pl.BlockSpec((blk, EMB), lambda i: (i, 0)),
            pl.BlockSpec((blk, EMB), lambda i: (i, 0)),
            w_spec, b_spec, w_spec,
            pl.BlockSpec((1, 2), lambda i: (0, 0)),
            pl.BlockSpec((EMB, C), lambda i: (0, 0)),
            pl.BlockSpec((1, C), lambda i: (0, 0)),
        ],
        out_specs=pl.BlockSpec((blk, C), lambda i: (i, 0)),
        out_shape=jax.ShapeDtypeStruct((N, C), jnp.float32),
    )(ne0, ne1, vf, Wl0, bl0.reshape(1, EMB), W4, w01,
      Wout, bout.reshape(1, C))


_NC = 2
_NS = 16
_NW = _NC * _NS


def _k2_sc(A, B, src, dst):
    per_w = E // _NW
    ch = 128
    nfull = per_w // ch
    tail = per_w - nfull * ch
    mesh = plsc.VectorSubcoreMesh(core_axis_name="c", subcore_axis_name="s")

    @functools.partial(
        pl.kernel,
        out_type=jax.ShapeDtypeStruct((E, EMB), jnp.float32),
        mesh=mesh,
        scratch_types=[
            pltpu.VMEM((ch,), jnp.int32),
            pltpu.VMEM((ch,), jnp.int32),
            pltpu.VMEM((ch, EMB), jnp.float32),
            pltpu.VMEM((ch, EMB), jnp.float32),
            pltpu.SemaphoreType.DMA,
            pltpu.SemaphoreType.DMA,
        ],
    )
    def k(a_hbm, b_hbm, src_hbm, dst_hbm, out_hbm,
          sidx, didx, bufa, bufb, sema, semb):
        wid = lax.axis_index("s") * _NC + lax.axis_index("c")
        base = wid * per_w

        def do_chunk(off, n):
            off = pl.multiple_of(off, 8)
            si = sidx if n == ch else sidx.at[pl.ds(0, n)]
            di = didx if n == ch else didx.at[pl.ds(0, n)]
            ba = bufa if n == ch else bufa.at[pl.ds(0, n)]
            bb = bufb if n == ch else bufb.at[pl.ds(0, n)]
            pltpu.sync_copy(src_hbm.at[pl.ds(off, n)], si)
            pltpu.sync_copy(dst_hbm.at[pl.ds(off, n)], di)
            da = pltpu.async_copy(a_hbm.at[si], ba, sema)
            db = pltpu.async_copy(b_hbm.at[di], bb, semb)
            da.wait()
            db.wait()

            def addrow(r, carry):
                for col in range(EMB // 16):
                    sl = pl.ds(col * 16, 16)
                    bufa[r, sl] = bufa[r, sl] + bufb[r, sl]
                return carry

            lax.fori_loop(0, n, addrow, 0)
            pltpu.sync_copy(ba, out_hbm.at[pl.ds(off, n)])

        def body(kk, carry):
            do_chunk(base + kk * ch, ch)
            return carry

        lax.fori_loop(0, nfull, body, 0)
        if tail:
            do_chunk(base + nfull * ch, tail)

    return k(A, B, src, dst)


def _k8_sc(P, dst):
    per_w = E // _NW
    ch = 128
    nfull = per_w // ch
    tail = per_w - nfull * ch
    rows_per_tile = N // _NS
    zch = 125
    mesh = plsc.VectorSubcoreMesh(core_axis_name="c", subcore_axis_name="s")

    @functools.partial(
        pl.kernel,
        out_type=jax.ShapeDtypeStruct((_NC, N, EMB), jnp.float32),
        mesh=mesh,
        scratch_types=[
            pltpu.VMEM((ch,), jnp.int32),
            pltpu.VMEM((ch, EMB), jnp.float32),
            pltpu.VMEM((zch, EMB), jnp.float32),
            pltpu.VMEM_SHARED((N, EMB), jnp.float32),
            pltpu.SemaphoreType.DMA,
        ],
    )
    def k(p_hbm, dst_hbm, out_hbm, didx, rows, zbuf, accum, sem):
        c = lax.axis_index("c")
        s = lax.axis_index("s")
        wid = s * _NC + c
        base = wid * per_w
        tbase = s * rows_per_tile

        def zrow(r, carry):
            for col in range(EMB // 16):
                zbuf[r, pl.ds(col * 16, 16)] = jnp.zeros((16,), jnp.float32)
            return carry

        lax.fori_loop(0, zch, zrow, 0)

        def zcopy(kk, carry):
            pltpu.sync_copy(zbuf, accum.at[pl.ds(tbase + kk * zch, zch)])
            return carry

        lax.fori_loop(0, rows_per_tile // zch, zcopy, 0)
        plsc.subcore_barrier()

        def do_chunk(off, n):
            off = pl.multiple_of(off, 8)
            di = didx if n == ch else didx.at[pl.ds(0, n)]
            rw = rows if n == ch else rows.at[pl.ds(0, n)]
            pltpu.sync_copy(dst_hbm.at[pl.ds(off, n)], di)
            pltpu.sync_copy(p_hbm.at[pl.ds(off, n)], rw)
            pltpu.sync_copy(rw, accum.at[di], add=True)

        def body(kk, carry):
            do_chunk(base + kk * ch, ch)
            return carry

        lax.fori_loop(0, nfull, body, 0)
        if tail:
            do_chunk(base + nfull * ch, tail)
        plsc.subcore_barrier()

        @pl.when(s < _NS - 1)
        def _copy_main():
            o = pl.multiple_of(s * 632, 8)
            pltpu.sync_copy(accum.at[pl.ds(o, 632)],
                            out_hbm.at[c, pl.ds(o, 632)])

        @pl.when(s == _NS - 1)
        def _copy_tail():
            pltpu.sync_copy(accum.at[pl.ds(9480, 520)],
                            out_hbm.at[c, pl.ds(9480, 520)])

    return k(P, dst)


_RB = 10000
_BLK = 1024
_ALLOC = 48 * 1024


def _k5a_sc(lsrc, ldst):
    SCAN = 2000
    per_tile = ELG // _NS
    nsub = per_tile // SCAN
    mesh = plsc.VectorSubcoreMesh(core_axis_name="c", subcore_axis_name="s")

    @functools.partial(
        pl.kernel,
        out_type=[
            jax.ShapeDtypeStruct((_NC * _NS * _ALLOC,), jnp.int32),
            jax.ShapeDtypeStruct((_NC * _NS * _ALLOC,), jnp.int32),
            jax.ShapeDtypeStruct((_NC * _NS * _ALLOC,), jnp.int32),
            jax.ShapeDtypeStruct((_NC * _NS * 256,), jnp.int32),
        ],
        mesh=mesh,
        compiler_params=pltpu.CompilerParams(needs_layout_passes=False),
        scratch_types=[
            pltpu.VMEM((SCAN,), jnp.int32),
            pltpu.VMEM((SCAN,), jnp.int32),
            pltpu.VMEM((_BLK + 16,), jnp.int32),
            pltpu.VMEM((_BLK + 16,), jnp.int32),
            pltpu.VMEM((_BLK + 16,), jnp.int32),
            pltpu.VMEM((256,), jnp.int32),
        ],
    )
    def k(lsrc_hbm, ldst_hbm, dlist, slist, jlist, hist,
          lds, lss, dblk, sblk, jblk, histv):
        c = lax.axis_index("c")
        s = lax.axis_index("s")
        iota16 = lax.iota(jnp.int32, 16)
        ones16 = jnp.ones((16,), jnp.int32)
        dump16 = jnp.full((16,), _RB, jnp.int32)
        zero16 = jnp.zeros((16,), jnp.int32)
        scan_base = s * per_tile

        def hz(i, cc):
            histv[pl.ds(i * 16, 16)] = zero16
            return cc

        lax.fori_loop(0, 16, hz, 0)

        def h_sub(i, cc):
            off = pl.multiple_of(scan_base + i * SCAN, 8)
            pltpu.sync_copy(ldst_hbm.at[pl.ds(off, SCAN)], lds)

            def h_step(t, cc2):
                v = lds[pl.ds(t * 16, 16)]
                b = v // _RB
                plsc.addupdate_scatter(histv, [b * 16 + iota16], ones16)
                return cc2

            lax.fori_loop(0, SCAN // 16, h_step, 0)
            return cc

        lax.fori_loop(0, nsub, h_sub, 0)
        ho = pl.multiple_of((c * _NS + s) * 256, 8)
        pltpu.sync_copy(histv, hist.at[pl.ds(ho, 256)])

        bases = []
        nblocks = []
        acc = jnp.int32(0)
        for p in range(8):
            cnt_p = jnp.sum(histv[pl.ds((2 * p + c) * 16, 16)])
            bases.append(acc)
            nb = cnt_p // 1009 + 1
            nblocks.append(nb)
            acc = acc + nb * _BLK

        def blk_init():
            def bi(g, cc):
                sl = pl.ds(g * 16, 16)
                dblk[sl] = dump16
                sblk[sl] = zero16
                jblk[sl] = zero16
                return cc

            lax.fori_loop(0, _BLK // 16, bi, 0)

        for p in range(8):
            base_p = bases[p]
            nb_p = nblocks[p]
            tgt_b = jnp.int32(2 * p) + c
            lo = tgt_b * _RB
            blk_init()

            def flush(fl, dst_off):
                o = pl.multiple_of((c * _NS + s) * _ALLOC + dst_off, 8)
                pltpu.sync_copy(dblk.at[pl.ds(0, _BLK)],
                                dlist.at[pl.ds(o, _BLK)])
                pltpu.sync_copy(sblk.at[pl.ds(0, _BLK)],
                                slist.at[pl.ds(o, _BLK)])
                pltpu.sync_copy(jblk.at[pl.ds(0, _BLK)],
                                jlist.at[pl.ds(o, _BLK)])

            def s2_sub(i, carry):
                fill, flushed = carry
                off = pl.multiple_of(scan_base + i * SCAN, 8)
                pltpu.sync_copy(ldst_hbm.at[pl.ds(off, SCAN)], lds)
                pltpu.sync_copy(lsrc_hbm.at[pl.ds(off, SCAN)], lss)

                def s2_step(t, carry2):
                    fill2, flushed2 = carry2

                    def do_flush(args):
                        f2, fl2 = args
                        flush(f2, base_p + fl2)
                        blk_init()
                        return jnp.int32(0), fl2 + _BLK

                    def no_flush(args):
                        return args

                    fill3, flushed3 = lax.cond(
                        fill2 > _BLK - 16, do_flush, no_flush,
                        (fill2, flushed2))

                    v = lds[pl.ds(t * 16, 16)]
                    vs = lss[pl.ds(t * 16, 16)]
                    b = v // _RB
                    d = b - tgt_b
                    in01 = 1 - jnp.minimum(d * d, 1)
                    ranks = plsc.cumsum(in01)
                    tgt = in01 * (fill3 + ranks - 1) + (1 - in01) * (_BLK + iota16)
                    jv = off + t * 16 + iota16
                    plsc.store_scatter(dblk, [tgt], v - lo)
                    plsc.store_scatter(sblk, [tgt], vs)
                    plsc.store_scatter(jblk, [tgt], jv)
                    return fill3 + jnp.max(ranks), flushed3

                return lax.fori_loop(0, SCAN // 16, s2_step, (fill, flushed))

            fill, flushed = lax.fori_loop(0, nsub, s2_sub,
                                          (jnp.int32(0), jnp.int32(0)))
            flush(fill, base_p + flushed)
            blk_init()
            done = flushed // _BLK + 1

            def padblk(kk, cc):
                flush(jnp.int32(0), base_p + (done + kk) * _BLK)
                return cc

            lax.fori_loop(0, nb_p - done, padblk, 0)

    return k(lsrc, ldst)


def _k5b_sc(M, G, dlist, slist, jlist, hist):
    CH = 128
    rows_pt = _RB // _NS
    mesh = plsc.VectorSubcoreMesh(core_axis_name="c", subcore_axis_name="s")

    @functools.partial(
        pl.kernel,
        out_type=jax.ShapeDtypeStruct((E, EMB), jnp.float32),
        mesh=mesh,
        compiler_params=pltpu.CompilerParams(needs_layout_passes=False),
        scratch_types=[
            pltpu.VMEM((CH,), jnp.int32),
            pltpu.VMEM((CH,), jnp.int32),
            pltpu.VMEM((CH,), jnp.int32),
            pltpu.VMEM((CH, EMB), jnp.float32),
            pltpu.VMEM((CH, EMB), jnp.float32),
            pltpu.VMEM((25, EMB), jnp.float32),
            pltpu.VMEM((256,), jnp.int32),
            pltpu.VMEM_SHARED((_RB + 8, EMB), jnp.float32),
            pltpu.SemaphoreType.DMA,
            pltpu.SemaphoreType.DMA,
        ],
    )
    def k(m_hbm, g_hbm, dlist_hbm, slist_hbm, jlist_hbm, hist_hbm, out_hbm,
          didx, sidx, jidx, mbuf, gbuf, zbuf, histv, accum, sem1, sem2):
        c = lax.axis_index("c")
        s = lax.axis_index("s")

        ho = pl.multiple_of((c * _NS + s) * 256, 8)
        pltpu.sync_copy(hist_hbm.at[pl.ds(ho, 256)], histv)

        bases = []
        nblocks = []
        acc = jnp.int32(0)
        for p in range(8):
            cnt_p = jnp.sum(histv[pl.ds((2 * p + c) * 16, 16)])
            bases.append(acc)
            nb = cnt_p // 1009 + 1
            nblocks.append(nb)
            acc = acc + nb * _BLK

        iota16 = lax.iota(jnp.int32, 16)
        zf16 = jnp.zeros((16,), jnp.float32)

        def zrow(r, carry):
            rfull = jnp.full((16,), r, jnp.int32)
            for col in range(EMB // 16):
                plsc.store_scatter(zbuf, [rfull, col * 16 + iota16], zf16)
            return carry

        lax.fori_loop(0, 25, zrow, 0)

        for p in range(8):
            lo = (2 * p + c) * _RB

            def zc(kk, cc):
                pltpu.sync_copy(zbuf, accum.at[pl.ds(s * rows_pt + kk * 25, 25)])
                return cc

            lax.fori_loop(0, rows_pt // 25, zc, 0)
            plsc.subcore_barrier()

            trip = nblocks[p] * (_BLK // CH)
            base_p = bases[p]

            def chunk(q, cc):
                loff = pl.multiple_of(
                    (c * _NS + s) * _ALLOC + base_p + q * CH, 8)
                pltpu.sync_copy(dlist_hbm.at[pl.ds(loff, CH)], didx)
                pltpu.sync_copy(slist_hbm.at[pl.ds(loff, CH)], sidx)
                pltpu.sync_copy(jlist_hbm.at[pl.ds(loff, CH)], jidx)
                dm = pltpu.async_copy(m_hbm.at[sidx], mbuf, sem1)
                dg = pltpu.async_copy(g_hbm.at[jidx], gbuf, sem2)
                dm.wait()
                dg.wait()

                def crow(r, cc2):
                    rfull = jnp.full((16,), r, jnp.int32)
                    for col in range(EMB // 16):
                        ci = col * 16 + iota16
                        m = plsc.load_gather(mbuf, [rfull, ci])
                        g = plsc.load_gather(gbuf, [rfull, ci])
                        plsc.store_scatter(mbuf, [rfull, ci],
                                           jnp.maximum(m + g, 0.0))
                    return cc2

                lax.fori_loop(0, CH, crow, 0)
                pltpu.sync_copy(mbuf, accum.at[didx], add=True)
                return cc

            lax.fori_loop(0, trip, chunk, 0)
            plsc.subcore_barrier()

            @pl.when(s < _NS - 1)
            def _copy_main():
                o = pl.multiple_of(s * 632, 8)
                pltpu.sync_copy(accum.at[pl.ds(o, 632)],
                                out_hbm.at[pl.ds(lo + o, 632)])

            @pl.when(s == _NS - 1)
            def _copy_tail():
                pltpu.sync_copy(accum.at[pl.ds(9480, 520)],
                                out_hbm.at[pl.ds(lo + 9480, 520)])

    return k(M, G, dlist, slist, jlist, hist)


def kernel(x, edge_index, edge_attr, lg_edge_index, lg_edge_attr,
           Wf, bf, Wm, bm, W1, b1, Wmsg,
           Wr0a, br0a, Wr0b, br0b, Wr1a, br1a, Wr1b, br1b,
           Wr2a, br2a, Wr2b, br2b,
           W2, b2, We, be, Wl0, bl0, W4, aggr, Wout, bout):
    src = edge_index[0]
    dst = edge_index[1]
    lsrc = lg_edge_index[0]
    ldst = lg_edge_index[1]

    Wm_a = Wm[:EMB]
    Wm_b = Wm[EMB:2 * EMB]
    Wm_e = Wm[2 * EMB:]
    Wmsg_m = Wmsg[:EMB]
    Wmsg_e = Wmsg[EMB:]

    vf, A, B = _k1(x, Wf, bf, Wm_a, Wm_b)

    ab = _k2_sc(A, B, src, dst)
    prev, M = _k3(ab, edge_attr, Wm_e, bm, W1, b1, Wmsg_m)

    G = _k4(lg_edge_attr, Wmsg_e)
    dlist, slist, jlist, hist = _k5a_sc(lsrc, ldst)
    new_msg = _k5b_sc(M, G, dlist, slist, jlist, hist)

    P = _k7(prev, new_msg, edge_attr,
            Wr0a, br0a, Wr0b, br0b, W2, b2,
            Wr1a, br1a, Wr1b, br1b, Wr2a, br2a, Wr2b, br2b,
            We, be)
    ne = _k8_sc(P, dst)

    w = jax.nn.softmax(aggr, axis=0).reshape(1, 2)
    return _k9(ne[0], ne[1], vf, Wl0, bl0, W4, w, Wout, bout)


# speedup vs baseline: 2.9513x; 2.9513x over previous
import functools

import jax
import jax.numpy as jnp
from jax import lax
from jax.experimental import pallas as pl
from jax.experimental.pallas import tpu as pltpu
from jax.experimental.pallas import tpu_sc as plsc

N = 10000
E = 160000
ELG = 640000
EMB = 128
DEA = 16
DLGA = 16
C = 16

_PREC = lax.Precision.DEFAULT


def _mm(a, b):
    return lax.dot_general(a, b, (((1,), (0,)), ((), ())),
                           preferred_element_type=jnp.float32,
                           precision=_PREC)


def _k1_body(x_ref, wf_ref, bf_ref, wma_ref, wmb_ref,
             vf_ref, a_ref, b_ref):
    vf = jnp.maximum(_mm(x_ref[...], wf_ref[...]) + bf_ref[...], 0.0)
    vf_ref[...] = vf
    a_ref[...] = _mm(vf, wma_ref[...])
    b_ref[...] = _mm(vf, wmb_ref[...])


def _k1(x, Wf, bf, Wm_a, Wm_b, blk=1000):
    grid = (N // blk,)
    w_spec = pl.BlockSpec((EMB, EMB), lambda i: (0, 0))
    return pl.pallas_call(
        _k1_body,
        grid=grid,
        in_specs=[
            pl.BlockSpec((blk, EMB), lambda i: (i, 0)),
            w_spec,
            pl.BlockSpec((1, EMB), lambda i: (0, 0)),
            w_spec,
            w_spec,
        ],
        out_specs=[
            pl.BlockSpec((blk, EMB), lambda i: (i, 0)),
            pl.BlockSpec((blk, EMB), lambda i: (i, 0)),
            pl.BlockSpec((blk, EMB), lambda i: (i, 0)),
        ],
        out_shape=[
            jax.ShapeDtypeStruct((N, EMB), jnp.float32),
            jax.ShapeDtypeStruct((N, EMB), jnp.float32),
            jax.ShapeDtypeStruct((N, EMB), jnp.float32),
        ],
    )(x, Wf, bf.reshape(1, EMB), Wm_a, Wm_b)


def _k3_body(ab_ref, ea_ref, wme_ref, bm_ref, w1_ref, b1_ref, wmm_ref,
             prev_ref, m_ref):
    me = (ab_ref[...] + _mm(jnp.maximum(ea_ref[...], 0.0), wme_ref[...])
          + bm_ref[...])
    prev_ref[...] = jnp.maximum(_mm(me, w1_ref[...]) + b1_ref[...], 0.0)
    m_ref[...] = _mm(me, wmm_ref[...])


def _k3(ab, edge_attr, Wm_e, bm, W1, b1, Wmsg_m, blk=2000):
    grid = (E // blk,)
    w_spec = pl.BlockSpec((EMB, EMB), lambda i: (0, 0))
    b_spec = pl.BlockSpec((1, EMB), lambda i: (0, 0))
    return pl.pallas_call(
        _k3_body,
        grid=grid,
        in_specs=[
            pl.BlockSpec((blk, EMB), lambda i: (i, 0)),
            pl.BlockSpec((blk, DEA), lambda i: (i, 0)),
            pl.BlockSpec((DEA, EMB), lambda i: (0, 0)),
            b_spec,
            w_spec,
            b_spec,
            w_spec,
        ],
        out_specs=[
            pl.BlockSpec((blk, EMB), lambda i: (i, 0)),
            pl.BlockSpec((blk, EMB), lambda i: (i, 0)),
        ],
        out_shape=[
            jax.ShapeDtypeStruct((E, EMB), jnp.float32),
            jax.ShapeDtypeStruct((E, EMB), jnp.float32),
        ],
    )(ab, edge_attr, Wm_e, bm.reshape(1, EMB), W1, b1.reshape(1, EMB), Wmsg_m)


def _k4_body(lga_ref, wme_ref, g_ref):
    g_ref[...] = _mm(lga_ref[...], wme_ref[...])


def _k4(lg_edge_attr, Wmsg_e, blk=4000):
    grid = (ELG // blk,)
    return pl.pallas_call(
        _k4_body,
        grid=grid,
        in_specs=[
            pl.BlockSpec((blk, DLGA), lambda i: (i, 0)),
            pl.BlockSpec((DLGA, EMB), lambda i: (0, 0)),
        ],
        out_specs=pl.BlockSpec((blk, EMB), lambda i: (i, 0)),
        out_shape=jax.ShapeDtypeStruct((ELG, EMB), jnp.float32),
    )(lg_edge_attr, Wmsg_e)


def _k7_body(prev_ref, nm_ref, ea_ref,
             wr0a_ref, br0a_ref, wr0b_ref, br0b_ref,
             w2_ref, b2_ref,
             wr1a_ref, br1a_ref, wr1b_ref, br1b_ref,
             wr2a_ref, br2a_ref, wr2b_ref, br2b_ref,
             we_ref, be_ref,
             p_ref):
    def res(h, wa, ba, wb, bb):
        h1 = jnp.maximum(_mm(h, wa[...]) + ba[...], 0.0)
        h2 = jnp.maximum(_mm(h1, wb[...]) + bb[...], 0.0)
        return h + h2

    h0 = prev_ref[...] + nm_ref[...]
    xh = res(h0, wr0a_ref, br0a_ref, wr0b_ref, br0b_ref)
    x1 = jnp.maximum(_mm(xh, w2_ref[...]) + b2_ref[...], 0.0)
    x3 = res(x1, wr1a_ref, br1a_ref, wr1b_ref, br1b_ref)
    x4 = res(x3, wr2a_ref, br2a_ref, wr2b_ref, br2b_ref)
    e_emb = _mm(ea_ref[...], we_ref[...]) + be_ref[...]
    p_ref[...] = e_emb * x4


def _k7(prev, new_msg, edge_attr,
        Wr0a, br0a, Wr0b, br0b, W2, b2,
        Wr1a, br1a, Wr1b, br1b, Wr2a, br2a, Wr2b, br2b,
        We, be, blk=2000):
    grid = (E // blk,)
    w_spec = pl.BlockSpec((EMB, EMB), lambda i: (0, 0))
    b_spec = pl.BlockSpec((1, EMB), lambda i: (0, 0))
    return pl.pallas_call(
        _k7_body,
        grid=grid,
        in_specs=[
            pl.BlockSpec((blk, EMB), lambda i: (i, 0)),
            pl.BlockSpec((blk, EMB), lambda i: (i, 0)),
            pl.BlockSpec((blk, DEA), lambda i: (i, 0)),
            w_spec, b_spec, w_spec, b_spec,
            w_spec, b_spec,
            w_spec, b_spec, w_spec, b_spec,
            w_spec, b_spec, w_spec, b_spec,
            pl.BlockSpec((DEA, EMB), lambda i: (0, 0)), b_spec,
        ],
        out_specs=pl.BlockSpec((blk, EMB), lambda i: (i, 0)),
        out_shape=jax.ShapeDtypeStruct((E, EMB), jnp.float32),
    )(prev, new_msg, edge_attr,
      Wr0a, br0a.reshape(1, EMB), Wr0b, br0b.reshape(1, EMB),
      W2, b2.reshape(1, EMB),
      Wr1a, br1a.reshape(1, EMB), Wr1b, br1b.reshape(1, EMB),
      Wr2a, br2a.reshape(1, EMB), Wr2b, br2b.reshape(1, EMB),
      We, be.reshape(1, EMB))


def _k9_body(ne0_ref, ne1_ref, vf_ref, wl0_ref, bl0_ref, w4_ref, w_ref,
             wout_ref, bout_ref, out_ref):
    ne = ne0_ref[...] + ne1_ref[...]
    o1 = jnp.maximum(_mm(ne, wl0_ref[...]) + bl0_ref[...], 0.0)
    o2 = jnp.maximum(_mm(o1, wl0_ref[...]) + bl0_ref[...], 0.0)
    o3 = jnp.maximum(_mm(o2, wl0_ref[...]) + bl0_ref[...], 0.0)
    x5 = _mm(o3, w4_ref[...])
    w0 = w_ref[0, 0]
    w1 = w_ref[0, 1]
    agg = w0 * vf_ref[...] + w1 * x5
    logits = _mm(agg, wout_ref[...]) + bout_ref[...]
    m = jnp.max(logits, axis=-1, keepdims=True)
    s = logits - m
    lse = jnp.log(jnp.sum(jnp.exp(s), axis=-1, keepdims=True))
    out_ref[...] = s - lse


def _k9(ne0, ne1, vf, Wl0, bl0, W4, w01, Wout, bout, blk=1000):
    grid = (N // blk,)
    w_spec = pl.BlockSpec((EMB, EMB), lambda i: (0, 0))
    b_spec = pl.BlockSpec((1, EMB), lambda i: (0, 0))
    return pl.pallas_call(
        _k9_body,
        grid=grid,
        in_specs=[
            pl.BlockSpec((blk, EMB), lambda i: (i, 0)),
            pl.BlockSpec((blk, EMB), lambda i: (i, 0)),
            pl.BlockSpec((blk, EMB), lambda i: (i, 0)),
            w_spec, b_spec, w_spec,
            pl.BlockSpec((1, 2), lambda i: (0, 0)),
            pl.BlockSpec((EMB, C), lambda i: (0, 0)),
            pl.BlockSpec((1, C), lambda i: (0, 0)),
        ],
        out_specs=pl.BlockSpec((blk, C), lambda i: (i, 0)),
        out_shape=jax.ShapeDtypeStruct((N, C), jnp.float32),
    )(ne0, ne1, vf, Wl0, bl0.reshape(1, EMB), W4, w01,
      Wout, bout.reshape(1, C))


_NC = 2
_NS = 16
_NW = _NC * _NS


def _k2_sc(A, B, src, dst):
    per_w = E // _NW
    ch = 128
    nfull = per_w // ch
    tail = per_w - nfull * ch
    mesh = plsc.VectorSubcoreMesh(core_axis_name="c", subcore_axis_name="s")

    @functools.partial(
        pl.kernel,
        out_type=jax.ShapeDtypeStruct((E, EMB), jnp.float32),
        mesh=mesh,
        scratch_types=[
            pltpu.VMEM((ch,), jnp.int32),
            pltpu.VMEM((ch,), jnp.int32),
            pltpu.VMEM((ch, EMB), jnp.float32),
            pltpu.VMEM((ch, EMB), jnp.float32),
            pltpu.SemaphoreType.DMA,
            pltpu.SemaphoreType.DMA,
        ],
    )
    def k(a_hbm, b_hbm, src_hbm, dst_hbm, out_hbm,
          sidx, didx, bufa, bufb, sema, semb):
        wid = lax.axis_index("s") * _NC + lax.axis_index("c")
        base = wid * per_w

        def do_chunk(off, n):
            off = pl.multiple_of(off, 8)
            si = sidx if n == ch else sidx.at[pl.ds(0, n)]
            di = didx if n == ch else didx.at[pl.ds(0, n)]
            ba = bufa if n == ch else bufa.at[pl.ds(0, n)]
            bb = bufb if n == ch else bufb.at[pl.ds(0, n)]
            pltpu.sync_copy(src_hbm.at[pl.ds(off, n)], si)
            pltpu.sync_copy(dst_hbm.at[pl.ds(off, n)], di)
            da = pltpu.async_copy(a_hbm.at[si], ba, sema)
            db = pltpu.async_copy(b_hbm.at[di], bb, semb)
            da.wait()
            db.wait()

            def addrow(r, carry):
                for col in range(EMB // 16):
                    sl = pl.ds(col * 16, 16)
                    bufa[r, sl] = bufa[r, sl] + bufb[r, sl]
                return carry

            lax.fori_loop(0, n, addrow, 0)
            pltpu.sync_copy(ba, out_hbm.at[pl.ds(off, n)])

        def body(kk, carry):
            do_chunk(base + kk * ch, ch)
            return carry

        lax.fori_loop(0, nfull, body, 0)
        if tail:
            do_chunk(base + nfull * ch, tail)

    return k(A, B, src, dst)


def _k8_sc(P, dst):
    per_w = E // _NW
    ch = 128
    nfull = per_w // ch
    tail = per_w - nfull * ch
    rows_per_tile = N // _NS
    zch = 125
    mesh = plsc.VectorSubcoreMesh(core_axis_name="c", subcore_axis_name="s")

    @functools.partial(
        pl.kernel,
        out_type=jax.ShapeDtypeStruct((_NC, N, EMB), jnp.float32),
        mesh=mesh,
        scratch_types=[
            pltpu.VMEM((ch,), jnp.int32),
            pltpu.VMEM((ch, EMB), jnp.float32),
            pltpu.VMEM((zch, EMB), jnp.float32),
            pltpu.VMEM_SHARED((N, EMB), jnp.float32),
            pltpu.SemaphoreType.DMA,
        ],
    )
    def k(p_hbm, dst_hbm, out_hbm, didx, rows, zbuf, accum, sem):
        c = lax.axis_index("c")
        s = lax.axis_index("s")
        wid = s * _NC + c
        base = wid * per_w
        tbase = s * rows_per_tile

        def zrow(r, carry):
            for col in range(EMB // 16):
                zbuf[r, pl.ds(col * 16, 16)] = jnp.zeros((16,), jnp.float32)
            return carry

        lax.fori_loop(0, zch, zrow, 0)

        def zcopy(kk, carry):
            pltpu.sync_copy(zbuf, accum.at[pl.ds(tbase + kk * zch, zch)])
            return carry

        lax.fori_loop(0, rows_per_tile // zch, zcopy, 0)
        plsc.subcore_barrier()

        def do_chunk(off, n):
            off = pl.multiple_of(off, 8)
            di = didx if n == ch else didx.at[pl.ds(0, n)]
            rw = rows if n == ch else rows.at[pl.ds(0, n)]
            pltpu.sync_copy(dst_hbm.at[pl.ds(off, n)], di)
            pltpu.sync_copy(p_hbm.at[pl.ds(off, n)], rw)
            pltpu.sync_copy(rw, accum.at[di], add=True)

        def body(kk, carry):
            do_chunk(base + kk * ch, ch)
            return carry

        lax.fori_loop(0, nfull, body, 0)
        if tail:
            do_chunk(base + nfull * ch, tail)
        plsc.subcore_barrier()

        @pl.when(s < _NS - 1)
        def _copy_main():
            o = pl.multiple_of(s * 632, 8)
            pltpu.sync_copy(accum.at[pl.ds(o, 632)],
                            out_hbm.at[c, pl.ds(o, 632)])

        @pl.when(s == _NS - 1)
        def _copy_tail():
            pltpu.sync_copy(accum.at[pl.ds(9480, 520)],
                            out_hbm.at[c, pl.ds(9480, 520)])

    return k(P, dst)


def _k5g_sc(M, G, lsrc):
    per_w = ELG // _NW
    ch = 384
    nfull = per_w // ch
    tail = per_w - nfull * ch
    mesh = plsc.VectorSubcoreMesh(core_axis_name="c", subcore_axis_name="s")

    @functools.partial(
        pl.kernel,
        out_type=jax.ShapeDtypeStruct((ELG, EMB), jnp.float32),
        mesh=mesh,
        scratch_types=[
            pltpu.VMEM((ch,), jnp.int32),
            pltpu.VMEM((ch, EMB), jnp.float32),
            pltpu.VMEM((ch, EMB), jnp.float32),
            pltpu.SemaphoreType.DMA,
            pltpu.SemaphoreType.DMA,
        ],
    )
    def k(m_hbm, g_hbm, lsrc_hbm, out_hbm, sidx, mbuf, gbuf, sema, semb):
        wid = lax.axis_index("s") * _NC + lax.axis_index("c")
        base = wid * per_w

        def do_chunk(off, n):
            off = pl.multiple_of(off, 8)
            si = sidx if n == ch else sidx.at[pl.ds(0, n)]
            mb = mbuf if n == ch else mbuf.at[pl.ds(0, n)]
            gb = gbuf if n == ch else gbuf.at[pl.ds(0, n)]
            pltpu.sync_copy(lsrc_hbm.at[pl.ds(off, n)], si)
            dm = pltpu.async_copy(m_hbm.at[si], mb, sema)
            dg = pltpu.async_copy(g_hbm.at[pl.ds(off, n)], gb, semb)
            dm.wait()
            dg.wait()

            def addrow(r, carry):
                for col in range(EMB // 16):
                    sl = pl.ds(col * 16, 16)
                    mbuf[r, sl] = jnp.maximum(mbuf[r, sl] + gbuf[r, sl], 0.0)
                return carry

            lax.fori_loop(0, n, addrow, 0)
            pltpu.sync_copy(mb, out_hbm.at[pl.ds(off, n)])

        def body(kk, carry):
            do_chunk(base + kk * ch, ch)
            return carry

        lax.fori_loop(0, nfull, body, 0)
        if tail:
            do_chunk(base + nfull * ch, tail)

    return k(M, G, lsrc)


def kernel(x, edge_index, edge_attr, lg_edge_index, lg_edge_attr,
           Wf, bf, Wm, bm, W1, b1, Wmsg,
           Wr0a, br0a, Wr0b, br0b, Wr1a, br1a, Wr1b, br1b,
           Wr2a, br2a, Wr2b, br2b,
           W2, b2, We, be, Wl0, bl0, W4, aggr, Wout, bout):
    src = edge_index[0]
    dst = edge_index[1]
    lsrc = lg_edge_index[0]
    ldst = lg_edge_index[1]

    Wm_a = Wm[:EMB]
    Wm_b = Wm[EMB:2 * EMB]
    Wm_e = Wm[2 * EMB:]
    Wmsg_m = Wmsg[:EMB]
    Wmsg_e = Wmsg[EMB:]

    vf, A, B = _k1(x, Wf, bf, Wm_a, Wm_b)

    ab = _k2_sc(A, B, src, dst)
    prev, M = _k3(ab, edge_attr, Wm_e, bm, W1, b1, Wmsg_m)

    G = _k4(lg_edge_attr, Wmsg_e)
    lm = _k5g_sc(M, G, lsrc)
    new_msg = jax.ops.segment_sum(lm, ldst, num_segments=E)

    P = _k7(prev, new_msg, edge_attr,
            Wr0a, br0a, Wr0b, br0b, W2, b2,
            Wr1a, br1a, Wr1b, br1b, Wr2a, br2a, Wr2b, br2b,
            We, be)
    ne = _k8_sc(P, dst)

    w = jax.nn.softmax(aggr, axis=0).reshape(1, 2)
    return _k9(ne[0], ne[1], vf, Wl0, bl0, W4, w, Wout, bout)


# speedup vs baseline: 2.9613x; 1.0034x over previous
import functools

import jax
import jax.numpy as jnp
from jax import lax
from jax.experimental import pallas as pl
from jax.experimental.pallas import tpu as pltpu
from jax.experimental.pallas import tpu_sc as plsc

N = 10000
E = 160000
ELG = 640000
EMB = 128
DEA = 16
DLGA = 16
C = 16

_PREC = lax.Precision.DEFAULT


def _mm(a, b):
    return lax.dot_general(a, b, (((1,), (0,)), ((), ())),
                           preferred_element_type=jnp.float32,
                           precision=_PREC)


def _k1_body(x_ref, wf_ref, bf_ref, wma_ref, wmb_ref,
             vf_ref, a_ref, b_ref):
    vf = jnp.maximum(_mm(x_ref[...], wf_ref[...]) + bf_ref[...], 0.0)
    vf_ref[...] = vf
    a_ref[...] = _mm(vf, wma_ref[...])
    b_ref[...] = _mm(vf, wmb_ref[...])


def _k1(x, Wf, bf, Wm_a, Wm_b, blk=1000):
    grid = (N // blk,)
    w_spec = pl.BlockSpec((EMB, EMB), lambda i: (0, 0))
    return pl.pallas_call(
        _k1_body,
        grid=grid,
        in_specs=[
            pl.BlockSpec((blk, EMB), lambda i: (i, 0)),
            w_spec,
            pl.BlockSpec((1, EMB), lambda i: (0, 0)),
            w_spec,
            w_spec,
        ],
        out_specs=[
            pl.BlockSpec((blk, EMB), lambda i: (i, 0)),
            pl.BlockSpec((blk, EMB), lambda i: (i, 0)),
            pl.BlockSpec((blk, EMB), lambda i: (i, 0)),
        ],
        out_shape=[
            jax.ShapeDtypeStruct((N, EMB), jnp.float32),
            jax.ShapeDtypeStruct((N, EMB), jnp.float32),
            jax.ShapeDtypeStruct((N, EMB), jnp.float32),
        ],
    )(x, Wf, bf.reshape(1, EMB), Wm_a, Wm_b)


def _k3_body(ab_ref, ea_ref, wme_ref, bm_ref, w1_ref, b1_ref, wmm_ref,
             prev_ref, m_ref):
    me = (ab_ref[...] + _mm(jnp.maximum(ea_ref[...], 0.0), wme_ref[...])
          + bm_ref[...])
    prev_ref[...] = jnp.maximum(_mm(me, w1_ref[...]) + b1_ref[...], 0.0)
    m_ref[...] = _mm(me, wmm_ref[...])


def _k3(ab, edge_attr, Wm_e, bm, W1, b1, Wmsg_m, blk=2000):
    grid = (E // blk,)
    w_spec = pl.BlockSpec((EMB, EMB), lambda i: (0, 0))
    b_spec = pl.BlockSpec((1, EMB), lambda i: (0, 0))
    return pl.pallas_call(
        _k3_body,
        grid=grid,
        in_specs=[
            pl.BlockSpec((blk, EMB), lambda i: (i, 0)),
            pl.BlockSpec((blk, DEA), lambda i: (i, 0)),
            pl.BlockSpec((DEA, EMB), lambda i: (0, 0)),
            b_spec,
            w_spec,
            b_spec,
            w_spec,
        ],
        out_specs=[
            pl.BlockSpec((blk, EMB), lambda i: (i, 0)),
            pl.BlockSpec((blk, EMB), lambda i: (i, 0)),
        ],
        out_shape=[
            jax.ShapeDtypeStruct((E, EMB), jnp.float32),
            jax.ShapeDtypeStruct((E, EMB), jnp.float32),
        ],
    )(ab, edge_attr, Wm_e, bm.reshape(1, EMB), W1, b1.reshape(1, EMB), Wmsg_m)


def _k4_body(lga_ref, wme_ref, g_ref):
    g_ref[...] = _mm(lga_ref[...], wme_ref[...])


def _k4(lg_edge_attr, Wmsg_e, blk=4000):
    grid = (ELG // blk,)
    return pl.pallas_call(
        _k4_body,
        grid=grid,
        in_specs=[
            pl.BlockSpec((blk, DLGA), lambda i: (i, 0)),
            pl.BlockSpec((DLGA, EMB), lambda i: (0, 0)),
        ],
        out_specs=pl.BlockSpec((blk, EMB), lambda i: (i, 0)),
        out_shape=jax.ShapeDtypeStruct((ELG, EMB), jnp.float32),
    )(lg_edge_attr, Wmsg_e)


def _k7_body(prev_ref, nm_ref, ea_ref,
             wr0a_ref, br0a_ref, wr0b_ref, br0b_ref,
             w2_ref, b2_ref,
             wr1a_ref, br1a_ref, wr1b_ref, br1b_ref,
             wr2a_ref, br2a_ref, wr2b_ref, br2b_ref,
             we_ref, be_ref,
             p_ref):
    def res(h, wa, ba, wb, bb):
        h1 = jnp.maximum(_mm(h, wa[...]) + ba[...], 0.0)
        h2 = jnp.maximum(_mm(h1, wb[...]) + bb[...], 0.0)
        return h + h2

    h0 = prev_ref[...] + nm_ref[...]
    xh = res(h0, wr0a_ref, br0a_ref, wr0b_ref, br0b_ref)
    x1 = jnp.maximum(_mm(xh, w2_ref[...]) + b2_ref[...], 0.0)
    x3 = res(x1, wr1a_ref, br1a_ref, wr1b_ref, br1b_ref)
    x4 = res(x3, wr2a_ref, br2a_ref, wr2b_ref, br2b_ref)
    e_emb = _mm(ea_ref[...], we_ref[...]) + be_ref[...]
    p_ref[...] = e_emb * x4


def _k7(prev, new_msg, edge_attr,
        Wr0a, br0a, Wr0b, br0b, W2, b2,
        Wr1a, br1a, Wr1b, br1b, Wr2a, br2a, Wr2b, br2b,
        We, be, blk=2000):
    grid = (E // blk,)
    w_spec = pl.BlockSpec((EMB, EMB), lambda i: (0, 0))
    b_spec = pl.BlockSpec((1, EMB), lambda i: (0, 0))
    return pl.pallas_call(
        _k7_body,
        grid=grid,
        in_specs=[
            pl.BlockSpec((blk, EMB), lambda i: (i, 0)),
            pl.BlockSpec((blk, EMB), lambda i: (i, 0)),
            pl.BlockSpec((blk, DEA), lambda i: (i, 0)),
            w_spec, b_spec, w_spec, b_spec,
            w_spec, b_spec,
            w_spec, b_spec, w_spec, b_spec,
            w_spec, b_spec, w_spec, b_spec,
            pl.BlockSpec((DEA, EMB), lambda i: (0, 0)), b_spec,
        ],
        out_specs=pl.BlockSpec((blk, EMB), lambda i: (i, 0)),
        out_shape=jax.ShapeDtypeStruct((E, EMB), jnp.float32),
    )(prev, new_msg, edge_attr,
      Wr0a, br0a.reshape(1, EMB), Wr0b, br0b.reshape(1, EMB),
      W2, b2.reshape(1, EMB),
      Wr1a, br1a.reshape(1, EMB), Wr1b, br1b.reshape(1, EMB),
      Wr2a, br2a.reshape(1, EMB), Wr2b, br2b.reshape(1, EMB),
      We, be.reshape(1, EMB))


def _k9_body(ne0_ref, ne1_ref, vf_ref, wl0_ref, bl0_ref, w4_ref, w_ref,
             wout_ref, bout_ref, out_ref):
    ne = ne0_ref[...] + ne1_ref[...]
    o1 = jnp.maximum(_mm(ne, wl0_ref[...]) + bl0_ref[...], 0.0)
    o2 = jnp.maximum(_mm(o1, wl0_ref[...]) + bl0_ref[...], 0.0)
    o3 = jnp.maximum(_mm(o2, wl0_ref[...]) + bl0_ref[...], 0.0)
    x5 = _mm(o3, w4_ref[...])
    w0 = w_ref[0, 0]
    w1 = w_ref[0, 1]
    agg = w0 * vf_ref[...] + w1 * x5
    logits = _mm(agg, wout_ref[...]) + bout_ref[...]
    m = jnp.max(logits, axis=-1, keepdims=True)
    s = logits - m
    lse = jnp.log(jnp.sum(jnp.exp(s), axis=-1, keepdims=True))
    out_ref[...] = s - lse


def _k9(ne0, ne1, vf, Wl0, bl0, W4, w01, Wout, bout, blk=1000):
    grid = (N // blk,)
    w_spec = pl.BlockSpec((EMB, EMB), lambda i: (0, 0))
    b_spec = pl.BlockSpec((1, EMB), lambda i: (0, 0))
    return pl.pallas_call(
        _k9_body,
        grid=grid,
        in_specs=[
            pl.BlockSpec((blk, EMB), lambda i: (i, 0)),
            pl.BlockSpec((blk, EMB), lambda i: (i, 0)),
            pl.BlockSpec((blk, EMB), lambda i: (i, 0)),
            w_spec, b_spec, w_spec,
            pl.BlockSpec((1, 2), lambda i: (0, 0)),
            pl.BlockSpec((EMB, C), lambda i: (0, 0)),
            pl.BlockSpec((1, C), lambda i: (0, 0)),
        ],
        out_specs=pl.BlockSpec((blk, C), lambda i: (i, 0)),
        out_shape=jax.ShapeDtypeStruct((N, C), jnp.float32),
    )(ne0, ne1, vf, Wl0, bl0.reshape(1, EMB), W4, w01,
      Wout, bout.reshape(1, C))


_NC = 2
_NS = 16
_NW = _NC * _NS


def _k2_sc(A, B, src, dst):
    per_w = E // _NW
    ch = 128
    nfull = per_w // ch
    tail = per_w - nfull * ch
    mesh = plsc.VectorSubcoreMesh(core_axis_name="c", subcore_axis_name="s")

    @functools.partial(
        pl.kernel,
        out_type=jax.ShapeDtypeStruct((E, EMB), jnp.float32),
        mesh=mesh,
        scratch_types=[
            pltpu.VMEM((ch,), jnp.int32),
            pltpu.VMEM((ch,), jnp.int32),
            pltpu.VMEM((ch, EMB), jnp.float32),
            pltpu.VMEM((ch, EMB), jnp.float32),
            pltpu.SemaphoreType.DMA,
            pltpu.SemaphoreType.DMA,
        ],
    )
    def k(a_hbm, b_hbm, src_hbm, dst_hbm, out_hbm,
          sidx, didx, bufa, bufb, sema, semb):
        wid = lax.axis_index("s") * _NC + lax.axis_index("c")
        base = wid * per_w

        def do_chunk(off, n):
            off = pl.multiple_of(off, 8)
            si = sidx if n == ch else sidx.at[pl.ds(0, n)]
            di = didx if n == ch else didx.at[pl.ds(0, n)]
            ba = bufa if n == ch else bufa.at[pl.ds(0, n)]
            bb = bufb if n == ch else bufb.at[pl.ds(0, n)]
            pltpu.sync_copy(src_hbm.at[pl.ds(off, n)], si)
            pltpu.sync_copy(dst_hbm.at[pl.ds(off, n)], di)
            da = pltpu.async_copy(a_hbm.at[si], ba, sema)
            db = pltpu.async_copy(b_hbm.at[di], bb, semb)
            da.wait()
            db.wait()

            def addrow(r, carry):
                for col in range(EMB // 16):
                    sl = pl.ds(col * 16, 16)
                    bufa[r, sl] = bufa[r, sl] + bufb[r, sl]
                return carry

            lax.fori_loop(0, n, addrow, 0)
            pltpu.sync_copy(ba, out_hbm.at[pl.ds(off, n)])

        def body(kk, carry):
            do_chunk(base + kk * ch, ch)
            return carry

        lax.fori_loop(0, nfull, body, 0)
        if tail:
            do_chunk(base + nfull * ch, tail)

    return k(A, B, src, dst)


def _k8_sc(P, dst):
    per_w = E // _NW
    ch = 128
    nfull = per_w // ch
    tail = per_w - nfull * ch
    rows_per_tile = N // _NS
    zch = 125
    mesh = plsc.VectorSubcoreMesh(core_axis_name="c", subcore_axis_name="s")

    @functools.partial(
        pl.kernel,
        out_type=jax.ShapeDtypeStruct((_NC, N, EMB), jnp.float32),
        mesh=mesh,
        scratch_types=[
            pltpu.VMEM((ch,), jnp.int32),
            pltpu.VMEM((ch, EMB), jnp.float32),
            pltpu.VMEM((zch, EMB), jnp.float32),
            pltpu.VMEM_SHARED((N, EMB), jnp.float32),
            pltpu.SemaphoreType.DMA,
        ],
    )
    def k(p_hbm, dst_hbm, out_hbm, didx, rows, zbuf, accum, sem):
        c = lax.axis_index("c")
        s = lax.axis_index("s")
        wid = s * _NC + c
        base = wid * per_w
        tbase = s * rows_per_tile

        def zrow(r, carry):
            for col in range(EMB // 16):
                zbuf[r, pl.ds(col * 16, 16)] = jnp.zeros((16,), jnp.float32)
            return carry

        lax.fori_loop(0, zch, zrow, 0)

        def zcopy(kk, carry):
            pltpu.sync_copy(zbuf, accum.at[pl.ds(tbase + kk * zch, zch)])
            return carry

        lax.fori_loop(0, rows_per_tile // zch, zcopy, 0)
        plsc.subcore_barrier()

        def do_chunk(off, n):
            off = pl.multiple_of(off, 8)
            di = didx if n == ch else didx.at[pl.ds(0, n)]
            rw = rows if n == ch else rows.at[pl.ds(0, n)]
            pltpu.sync_copy(dst_hbm.at[pl.ds(off, n)], di)
            pltpu.sync_copy(p_hbm.at[pl.ds(off, n)], rw)
            pltpu.sync_copy(rw, accum.at[di], add=True)

        def body(kk, carry):
            do_chunk(base + kk * ch, ch)
            return carry

        lax.fori_loop(0, nfull, body, 0)
        if tail:
            do_chunk(base + nfull * ch, tail)
        plsc.subcore_barrier()

        @pl.when(s < _NS - 1)
        def _copy_main():
            o = pl.multiple_of(s * 632, 8)
            pltpu.sync_copy(accum.at[pl.ds(o, 632)],
                            out_hbm.at[c, pl.ds(o, 632)])

        @pl.when(s == _NS - 1)
        def _copy_tail():
            pltpu.sync_copy(accum.at[pl.ds(9480, 520)],
                            out_hbm.at[c, pl.ds(9480, 520)])

    return k(P, dst)


def _k5g_sc(M, G, lsrc):
    per_w = ELG // _NW
    ch = 192
    nfull = per_w // ch
    tail = per_w - nfull * ch
    npairs = nfull // 2
    mesh = plsc.VectorSubcoreMesh(core_axis_name="c", subcore_axis_name="s")

    @functools.partial(
        pl.kernel,
        out_type=jax.ShapeDtypeStruct((ELG, EMB), jnp.float32),
        mesh=mesh,
        scratch_types=[
            pltpu.VMEM((ch,), jnp.int32),
            pltpu.VMEM((ch,), jnp.int32),
            pltpu.VMEM((ch, EMB), jnp.float32),
            pltpu.VMEM((ch, EMB), jnp.float32),
            pltpu.VMEM((ch, EMB), jnp.float32),
            pltpu.VMEM((ch, EMB), jnp.float32),
            pltpu.SemaphoreType.DMA,
            pltpu.SemaphoreType.DMA,
            pltpu.SemaphoreType.DMA,
            pltpu.SemaphoreType.DMA,
            pltpu.SemaphoreType.DMA,
            pltpu.SemaphoreType.DMA,
        ],
    )
    def k(m_hbm, g_hbm, lsrc_hbm, out_hbm,
          sia, sib, mba, gba, mbb, gbb,
          semma, semga, semoa, semmb, semgb, semob):
        wid = lax.axis_index("s") * _NC + lax.axis_index("c")
        base = wid * per_w

        def stage_issue(off, si, mb, gb, semm, semg):
            pltpu.async_copy(g_hbm.at[pl.ds(off, ch)], gb, semg)
            pltpu.sync_copy(lsrc_hbm.at[pl.ds(off, ch)], si)
            pltpu.async_copy(m_hbm.at[si], mb, semm)

        def wait_gathers(off, si, mb, gb, semm, semg):
            pltpu.make_async_copy(m_hbm.at[si], mb, semm).wait()
            pltpu.make_async_copy(g_hbm.at[pl.ds(off, ch)], gb, semg).wait()

        def compute(mb, gb):
            def addrow(r, carry):
                for col in range(EMB // 16):
                    sl = pl.ds(col * 16, 16)
                    mb[r, sl] = jnp.maximum(mb[r, sl] + gb[r, sl], 0.0)
                return carry

            lax.fori_loop(0, ch, addrow, 0)

        stage_issue(pl.multiple_of(base, 8), sia, mba, gba, semma, semga)
        stage_issue(pl.multiple_of(base + ch, 8), sib, mbb, gbb, semmb, semgb)

        def body(i, carry):
            offa = pl.multiple_of(base + (2 * i) * ch, 8)
            offb = pl.multiple_of(base + (2 * i + 1) * ch, 8)
            wait_gathers(offa, sia, mba, gba, semma, semga)
            compute(mba, gba)
            pltpu.async_copy(mba, out_hbm.at[pl.ds(offa, ch)], semoa)

            @pl.when(i < npairs - 1)
            def _refill_a():
                pltpu.make_async_copy(mba, out_hbm.at[pl.ds(offa, ch)],
                                      semoa).wait()
                stage_issue(pl.multiple_of(base + (2 * i + 2) * ch, 8),
                            sia, mba, gba, semma, semga)

            wait_gathers(offb, sib, mbb, gbb, semmb, semgb)
            compute(mbb, gbb)
            pltpu.async_copy(mbb, out_hbm.at[pl.ds(offb, ch)], semob)

            @pl.when(i < npairs - 1)
            def _refill_b():
                pltpu.make_async_copy(mbb, out_hbm.at[pl.ds(offb, ch)],
                                      semob).wait()
                stage_issue(pl.multiple_of(base + (2 * i + 3) * ch, 8),
                            sib, mbb, gbb, semmb, semgb)

            return carry

        lax.fori_loop(0, npairs, body, 0)
        o_last_a = pl.multiple_of(base + (nfull - 2) * ch, 8)
        o_last_b = pl.multiple_of(base + (nfull - 1) * ch, 8)
        pltpu.make_async_copy(mba, out_hbm.at[pl.ds(o_last_a, ch)],
                              semoa).wait()
        pltpu.make_async_copy(mbb, out_hbm.at[pl.ds(o_last_b, ch)],
                              semob).wait()

        if tail:
            toff = pl.multiple_of(base + nfull * ch, 8)
            sit = sia.at[pl.ds(0, tail)]
            mbt = mba.at[pl.ds(0, tail)]
            gbt = gba.at[pl.ds(0, tail)]
            pltpu.sync_copy(lsrc_hbm.at[pl.ds(toff, tail)], sit)
            da = pltpu.async_copy(m_hbm.at[sit], mbt, semma)
            db = pltpu.async_copy(g_hbm.at[pl.ds(toff, tail)], gbt, semga)
            da.wait()
            db.wait()

            def addrow_t(r, carry):
                for col in range(EMB // 16):
                    sl = pl.ds(col * 16, 16)
                    mba[r, sl] = jnp.maximum(mba[r, sl] + gba[r, sl], 0.0)
                return carry

            lax.fori_loop(0, tail, addrow_t, 0)
            pltpu.sync_copy(mbt, out_hbm.at[pl.ds(toff, tail)])

    return k(M, G, lsrc)


def kernel(x, edge_index, edge_attr, lg_edge_index, lg_edge_attr,
           Wf, bf, Wm, bm, W1, b1, Wmsg,
           Wr0a, br0a, Wr0b, br0b, Wr1a, br1a, Wr1b, br1b,
           Wr2a, br2a, Wr2b, br2b,
           W2, b2, We, be, Wl0, bl0, W4, aggr, Wout, bout):
    src = edge_index[0]
    dst = edge_index[1]
    lsrc = lg_edge_index[0]
    ldst = lg_edge_index[1]

    Wm_a = Wm[:EMB]
    Wm_b = Wm[EMB:2 * EMB]
    Wm_e = Wm[2 * EMB:]
    Wmsg_m = Wmsg[:EMB]
    Wmsg_e = Wmsg[EMB:]

    vf, A, B = _k1(x, Wf, bf, Wm_a, Wm_b)

    ab = _k2_sc(A, B, src, dst)
    prev, M = _k3(ab, edge_attr, Wm_e, bm, W1, b1, Wmsg_m)

    G = _k4(lg_edge_attr, Wmsg_e)
    lm = _k5g_sc(M, G, lsrc)
    new_msg = jax.ops.segment_sum(lm, ldst, num_segments=E)

    P = _k7(prev, new_msg, edge_attr,
            Wr0a, br0a, Wr0b, br0b, W2, b2,
            Wr1a, br1a, Wr1b, br1b, Wr2a, br2a, Wr2b, br2b,
            We, be)
    ne = _k8_sc(P, dst)

    w = jax.nn.softmax(aggr, axis=0).reshape(1, 2)
    return _k9(ne[0], ne[1], vf, Wl0, bl0, W4, w, Wout, bout)
